# Initial kernel scaffold; baseline (speedup 1.0000x reference)
import jax
import jax.numpy as jnp
from jax.experimental import pallas as pl


def kernel(x_M, x_E, x_S, ei_M_M, ei_M_E, ei_M_S, ei_E_S, ei_E_M, ei_S_M, ei_S_E, ei_S_S, ei_E_E, W_M_M, W_M_E, W_M_S, W_E_S, W_E_M, W_S_M, W_S_E, W_S_S, W_E_E, b_M_M, b_M_E, b_M_S, b_E_S, b_E_M, b_S_M, b_S_E, b_S_S, b_E_E):
    raise NotImplementedError("write your pallas kernel here")


# speedup vs baseline: 1.7191x; 1.7191x over previous
import jax
import jax.numpy as jnp
from jax import lax
from jax.experimental import pallas as pl
from jax.experimental.pallas import tpu as pltpu
from jax.experimental.pallas import tpu_sc as plsc

N = 50000
E = 64000
D = 128
NC = 2
NS = 16
L = 16
SB = 128
NSB = E // SB
ZN = 3136
EPT = E // NS
EH = EPT // 2
SBE = 80
NSUB = EH // SBE
CHUNK = 12800
ACC_ROWS = CHUNK + 32
WB = 40
NWIN = CHUNK // WB

REL_NAMES = ["M_M", "M_E", "M_S", "E_S", "E_M", "S_M", "S_E", "S_S", "E_E"]
SRC_SLOTS = {"M": ["M_M", "M_E", "M_S"], "E": ["E_S", "E_M", "E_E"],
             "S": ["S_M", "S_E", "S_S"]}
DST_RELS = {
    "M": ["M_M", "E_M", "S_M"],
    "E": ["M_E", "S_E", "E_E"],
    "S": ["M_S", "E_S", "S_S"],
}

_mesh = plsc.VectorSubcoreMesh(core_axis_name="c", subcore_axis_name="s",
                               num_cores=NC, num_subcores=NS)


def _num_subbatches(tid):
  return 31 + jnp.where(tid < NSB - 31 * NS, 1, 0)


def _rsqrt_newton(d):
  y = lax.bitcast_convert_type(
      jnp.int32(0x5F3759DF) - (lax.bitcast_convert_type(d, jnp.int32) >> 1),
      jnp.float32)
  for _ in range(3):
    y = y * (1.5 - 0.5 * d * y * y)
  return y


def _lane_bcast_dyn(v16, lane):
  idx = (jnp.zeros((L, 1), jnp.int32) + lane).astype(jnp.int32)
  return lax.gather(
      v16, idx,
      lax.GatherDimensionNumbers(offset_dims=(), collapsed_slice_dims=(0,),
                                 start_index_map=(0,)),
      (1,), mode=lax.GatherScatterMode.PROMISE_IN_BOUNDS)


def _degrees_body(*refs):
  src_refs = refs[:9]
  dst_refs = refs[9:18]
  od_refs = refs[18:27]
  id_refs = refs[27:36]
  (srcbuf, dstbuf, onesbuf, dbuf, zbuf, outdeg_sp, indeg_sp) = refs[36:]

  cid = lax.axis_index("c")
  tid = lax.axis_index("s")
  nb = _num_subbatches(tid)

  def _fill_ones(i, _):
    onesbuf[pl.ds(i * L, L)] = jnp.full((L,), 1.0, jnp.float32)
    return 0
  lax.fori_loop(0, SB // L, _fill_ones, 0)

  def _fill_zeros(i, _):
    zbuf[pl.ds(i * L, L)] = jnp.zeros((L,), jnp.float32)
    return 0
  lax.fori_loop(0, ZN // L, _fill_zeros, 0)

  r0 = pl.multiple_of(jnp.where(tid < NS - 1, tid * ZN, N - ZN), 8)

  for r in range(9):
    @pl.when(cid == r % NC)
    def _process():
      src_r = src_refs[r]
      dst_r = dst_refs[r]
      pltpu.sync_copy(zbuf, outdeg_sp.at[pl.ds(r0, ZN)])
      pltpu.sync_copy(zbuf, indeg_sp.at[pl.ds(r0, ZN)])
      plsc.subcore_barrier()

      def _hist(j, _):
        off = (tid + NS * j) * SB
        pltpu.sync_copy(src_r.at[pl.ds(off, SB)], srcbuf)
        pltpu.sync_copy(dst_r.at[pl.ds(off, SB)], dstbuf)
        pltpu.sync_copy(onesbuf, outdeg_sp.at[srcbuf], add=True)
        pltpu.sync_copy(onesbuf, indeg_sp.at[dstbuf], add=True)
        return 0
      lax.fori_loop(0, nb, _hist, 0)
      plsc.subcore_barrier()

      for hist_sp, out_hbm in ((outdeg_sp, od_refs[r]),
                               (indeg_sp, id_refs[r])):
        pltpu.sync_copy(hist_sp.at[pl.ds(r0, ZN)], dbuf)
        def _rs(i, _):
          d = jnp.maximum(dbuf[pl.ds(i * L, L)], 1.0)
          dbuf[pl.ds(i * L, L)] = _rsqrt_newton(d)
          return 0
        lax.fori_loop(0, ZN // L, _rs, 0)
        pltpu.sync_copy(dbuf, out_hbm.at[pl.ds(r0, ZN)])
      plsc.subcore_barrier()


def _degrees(src_list, dst_list):
  kern = pl.kernel(
      _degrees_body,
      out_type=[jax.ShapeDtypeStruct((N,), jnp.float32) for _ in range(18)],
      mesh=_mesh,
      scratch_types=[
          pltpu.VMEM((SB,), jnp.int32),
          pltpu.VMEM((SB,), jnp.int32),
          pltpu.VMEM((SB,), jnp.float32),
          pltpu.VMEM((ZN,), jnp.float32),
          pltpu.VMEM((ZN,), jnp.float32),
          pltpu.VMEM_SHARED((N,), jnp.float32),
          pltpu.VMEM_SHARED((N,), jnp.float32),
      ],
  )
  outs = kern(*src_list, *dst_list)
  return outs[:9], outs[9:]


def _agg_body(*refs):
  y_refs = dict(zip(REL_NAMES, refs[0:9]))
  esrc_refs = dict(zip(REL_NAMES, refs[9:18]))
  edst_refs = dict(zip(REL_NAMES, refs[18:27]))
  id_refs = dict(zip(REL_NAMES, refs[27:36]))
  bias_ref = refs[36]
  out_refs = {"M": refs[37], "E": refs[38], "S": refs[39]}
  (srcbuf, dstbuf, fdst2d, fdst1r, gsrc, g0, g1, irbuf, bbuf, gsem0, gsem1,
   ssem0, ssem1, acc_sp) = refs[40:]
  gbufs = (g0, g1)
  gsems = (gsem0, gsem1)
  ssems = (ssem0, ssem1)

  cid = lax.axis_index("c")
  tid = lax.axis_index("s")

  pltpu.sync_copy(bias_ref, bbuf)
  iota = lax.iota(jnp.int32, L)
  dummy_dst = CHUNK + iota * 2 + (tid & 1)

  nwin = (NWIN // NS) + jnp.where(tid < NWIN - (NWIN // NS) * NS, 1, 0)

  def _chunk(kk, _c):
    ci = cid + NC * kk
    lo = ci * CHUNK
    hi = jnp.minimum(lo + CHUNK, N)
    climit = hi - lo

    for dname in ["M", "E", "S"]:
      di = ["M", "E", "S"].index(dname)
      out = out_refs[dname]
      bvs = [bbuf[di, pl.ds(c8 * L, L)] for c8 in range(D // L)]

      for k, rname in enumerate(DST_RELS[dname]):
        esrc = esrc_refs[rname]
        edst = edst_refs[rname]
        y_hbm = y_refs[rname]
        ir_hbm = id_refs[rname]

        def _zfill(i, _):
          for c8 in range(D // L):
            g0[i, pl.ds(c8 * L, L)] = jnp.zeros((L,), jnp.float32)
          return 0
        lax.fori_loop(0, WB, _zfill, 0)

        def _zero(w, _):
          pltpu.sync_copy(g0.at[pl.ds(0, WB)],
                          acc_sp.at[pl.ds((tid + NS * w) * WB, WB)])
          return 0
        lax.fori_loop(0, nwin, _zero, 0)
        plsc.subcore_barrier()

        for h in range(2):
          eoff = tid * EPT + h * EH
          pltpu.sync_copy(esrc.at[pl.ds(eoff, EH)], srcbuf)
          pltpu.sync_copy(edst.at[pl.ds(eoff, EH)], dstbuf)

          def _mask(b, _):
            for g in range(SBE // L):
              dv = dstbuf[pl.ds(b * SBE + g * L, L)]
              m = (dv >= lo) & (dv < hi)
              fdst2d[b, pl.ds(g * L, L)] = jnp.where(m, dv - lo, dummy_dst)
            return 0
          lax.fori_loop(0, NSUB, _mask, 0)

          def _gs(b, _):
            for o in range(0, SBE, L):
              fdst1r[0, pl.ds(o, L)] = fdst2d[b, pl.ds(o, L)]
              gsrc[pl.ds(o, L)] = srcbuf[pl.ds(b * SBE + o, L)]
            pltpu.sync_copy(y_hbm.at[gsrc], g0)
            pltpu.sync_copy(g0, acc_sp.at[fdst1r.at[0]], add=True)
            return 0
          lax.fori_loop(0, NSUB, _gs, 0)
        plsc.subcore_barrier()

        def _wb(w, _):
          ls = pl.multiple_of((tid + NS * w) * WB, 8)

          @pl.when(ls < climit)
          def _do_wb():
            pltpu.sync_copy(acc_sp.at[pl.ds(ls, WB)], g0.at[pl.ds(0, WB)])
            pltpu.sync_copy(ir_hbm.at[pl.ds(lo + ls, WB)],
                            irbuf.at[pl.ds(0, WB)])
            if k > 0:
              pltpu.sync_copy(out.at[pl.ds(lo + ls, WB)], g1)

            def _row(i, _):
              gi = i >> 4
              li = i & (L - 1)
              irv = irbuf[pl.ds(gi * L, L)]
              s = _lane_bcast_dyn(irv, li)
              for c8 in range(D // L):
                v = g0[i, pl.ds(c8 * L, L)] * s
                if k > 0:
                  v = v + g1[i, pl.ds(c8 * L, L)]
                if k == 2:
                  v = jnp.maximum(v + bvs[c8], 0.0)
                g0[i, pl.ds(c8 * L, L)] = v
              return 0
            lax.fori_loop(0, WB, _row, 0)
            pltpu.sync_copy(g0.at[pl.ds(0, WB)], out.at[pl.ds(lo + ls, WB)])
          return 0
        lax.fori_loop(0, nwin, _wb, 0)
        plsc.subcore_barrier()
    return 0

  lax.fori_loop(0, 2, _chunk, 0)


def _aggregate(y_by_rel, src_list, dst_list, id_list, bias_sums):
  kern = pl.kernel(
      _agg_body,
      out_type=[jax.ShapeDtypeStruct((N, D), jnp.float32) for _ in range(3)],
      mesh=_mesh,
      scratch_types=[
          pltpu.VMEM((EH,), jnp.int32),
          pltpu.VMEM((EH,), jnp.int32),
          pltpu.VMEM((NSUB, SBE), jnp.int32),
          pltpu.VMEM((1, SBE), jnp.int32),
          pltpu.VMEM((SBE,), jnp.int32),
          pltpu.VMEM((SBE, D), jnp.float32),
          pltpu.VMEM((WB, D), jnp.float32),
          pltpu.VMEM((WB + 8, ), jnp.float32),
          pltpu.VMEM((3, D), jnp.float32),
          pltpu.SemaphoreType.DMA,
          pltpu.SemaphoreType.DMA,
          pltpu.SemaphoreType.DMA,
          pltpu.SemaphoreType.DMA,
          pltpu.VMEM_SHARED((ACC_ROWS, D), jnp.float32),
      ],
  )
  return kern(*[y_by_rel[n] for n in REL_NAMES], *src_list, *dst_list,
              *id_list, bias_sums)


def _mm_body(x_ref, w0_ref, w1_ref, w2_ref, d0_ref, d1_ref, d2_ref,
             o0_ref, o1_ref, o2_ref):
  x = x_ref[...]
  o0_ref[...] = jnp.dot(x * d0_ref[...][:, None], w0_ref[...],
                        preferred_element_type=jnp.float32)
  o1_ref[...] = jnp.dot(x * d1_ref[...][:, None], w1_ref[...],
                        preferred_element_type=jnp.float32)
  o2_ref[...] = jnp.dot(x * d2_ref[...][:, None], w2_ref[...],
                        preferred_element_type=jnp.float32)


def _matmul3(x, ws, ods):
  bm = 512
  grid = (pl.cdiv(N, bm),)
  wspec = pl.BlockSpec((D, D), lambda i: (0, 0))
  dspec = pl.BlockSpec((bm,), lambda i: (i,))
  ospec = pl.BlockSpec((bm, D), lambda i: (i, 0))
  return pl.pallas_call(
      _mm_body,
      grid=grid,
      in_specs=[pl.BlockSpec((bm, D), lambda i: (i, 0)),
                wspec, wspec, wspec, dspec, dspec, dspec],
      out_specs=[ospec, ospec, ospec],
      out_shape=[jax.ShapeDtypeStruct((N, D), jnp.float32) for _ in range(3)],
  )(x, *ws, *ods)


def kernel(x_M, x_E, x_S,
           ei_M_M, ei_M_E, ei_M_S, ei_E_S, ei_E_M, ei_S_M, ei_S_E, ei_S_S,
           ei_E_E,
           W_M_M, W_M_E, W_M_S, W_E_S, W_E_M, W_S_M, W_S_E, W_S_S, W_E_E,
           b_M_M, b_M_E, b_M_S, b_E_S, b_E_M, b_S_M, b_S_E, b_S_S, b_E_E):
  x = {"M": x_M, "E": x_E, "S": x_S}
  ei = {"M_M": ei_M_M, "M_E": ei_M_E, "M_S": ei_M_S, "E_S": ei_E_S,
        "E_M": ei_E_M, "S_M": ei_S_M, "S_E": ei_S_E, "S_S": ei_S_S,
        "E_E": ei_E_E}
  W = {"M_M": W_M_M, "M_E": W_M_E, "M_S": W_M_S, "E_S": W_E_S, "E_M": W_E_M,
       "S_M": W_S_M, "S_E": W_S_E, "S_S": W_S_S, "E_E": W_E_E}
  b = {"M_M": b_M_M, "M_E": b_M_E, "M_S": b_M_S, "E_S": b_E_S, "E_M": b_E_M,
       "S_M": b_S_M, "S_E": b_S_E, "S_S": b_S_S, "E_E": b_E_E}

  src_list = [ei[n][0] for n in REL_NAMES]
  dst_list = [ei[n][1] for n in REL_NAMES]
  od_list, id_list = _degrees(src_list, dst_list)
  od = dict(zip(REL_NAMES, od_list))

  y_by_rel = {}
  for s, slots in SRC_SLOTS.items():
    ys = _matmul3(x[s], [W[r] for r in slots], [od[r] for r in slots])
    for r, y in zip(slots, ys):
      y_by_rel[r] = y

  bias_sums = jnp.stack([b[DST_RELS[d][0]] + b[DST_RELS[d][1]]
                         + b[DST_RELS[d][2]] for d in ["M", "E", "S"]])

  return tuple(_aggregate(y_by_rel, src_list, dst_list, id_list,
                          bias_sums))


# speedup vs baseline: 2.1665x; 1.2602x over previous
import jax
import jax.numpy as jnp
from jax import lax
from jax.experimental import pallas as pl
from jax.experimental.pallas import tpu as pltpu
from jax.experimental.pallas import tpu_sc as plsc

N = 50000
E = 64000
D = 128
NC = 2
NS = 16
L = 16
SB = 128
NSB = E // SB
ZN = 3136
EPT = E // NS
EH = EPT // 2
SBE = 80
NSUB = EH // SBE
CHUNK = 12800
ACC_ROWS = CHUNK + 32
WB = 40
NWIN = CHUNK // WB

REL_NAMES = ["M_M", "M_E", "M_S", "E_S", "E_M", "S_M", "S_E", "S_S", "E_E"]
SRC_SLOTS = {"M": ["M_M", "M_E", "M_S"], "E": ["E_S", "E_M", "E_E"],
             "S": ["S_M", "S_E", "S_S"]}
DST_RELS = {
    "M": ["M_M", "E_M", "S_M"],
    "E": ["M_E", "S_E", "E_E"],
    "S": ["M_S", "E_S", "S_S"],
}

_mesh = plsc.VectorSubcoreMesh(core_axis_name="c", subcore_axis_name="s",
                               num_cores=NC, num_subcores=NS)


def _num_subbatches(tid):
  return 31 + jnp.where(tid < NSB - 31 * NS, 1, 0)


def _rsqrt_newton(d):
  y = lax.bitcast_convert_type(
      jnp.int32(0x5F3759DF) - (lax.bitcast_convert_type(d, jnp.int32) >> 1),
      jnp.float32)
  for _ in range(3):
    y = y * (1.5 - 0.5 * d * y * y)
  return y


def _lane_bcast_dyn(v16, lane):
  idx = (jnp.zeros((L, 1), jnp.int32) + lane).astype(jnp.int32)
  return lax.gather(
      v16, idx,
      lax.GatherDimensionNumbers(offset_dims=(), collapsed_slice_dims=(0,),
                                 start_index_map=(0,)),
      (1,), mode=lax.GatherScatterMode.PROMISE_IN_BOUNDS)


def _degrees_body(*refs):
  src_refs = refs[:9]
  dst_refs = refs[9:18]
  od_refs = refs[18:27]
  id_refs = refs[27:36]
  (srcbuf, dstbuf, onesbuf, dbuf, zbuf, outdeg_sp, indeg_sp) = refs[36:]

  cid = lax.axis_index("c")
  tid = lax.axis_index("s")
  nb = _num_subbatches(tid)

  def _fill_ones(i, _):
    onesbuf[pl.ds(i * L, L)] = jnp.full((L,), 1.0, jnp.float32)
    return 0
  lax.fori_loop(0, SB // L, _fill_ones, 0)

  def _fill_zeros(i, _):
    zbuf[pl.ds(i * L, L)] = jnp.zeros((L,), jnp.float32)
    return 0
  lax.fori_loop(0, ZN // L, _fill_zeros, 0)

  r0 = pl.multiple_of(jnp.where(tid < NS - 1, tid * ZN, N - ZN), 8)

  for r in range(9):
    @pl.when(cid == r % NC)
    def _process():
      src_r = src_refs[r]
      dst_r = dst_refs[r]
      pltpu.sync_copy(zbuf, outdeg_sp.at[pl.ds(r0, ZN)])
      pltpu.sync_copy(zbuf, indeg_sp.at[pl.ds(r0, ZN)])
      plsc.subcore_barrier()

      def _hist(j, _):
        off = (tid + NS * j) * SB
        pltpu.sync_copy(src_r.at[pl.ds(off, SB)], srcbuf)
        pltpu.sync_copy(dst_r.at[pl.ds(off, SB)], dstbuf)
        pltpu.sync_copy(onesbuf, outdeg_sp.at[srcbuf], add=True)
        pltpu.sync_copy(onesbuf, indeg_sp.at[dstbuf], add=True)
        return 0
      lax.fori_loop(0, nb, _hist, 0)
      plsc.subcore_barrier()

      for hist_sp, out_hbm in ((outdeg_sp, od_refs[r]),
                               (indeg_sp, id_refs[r])):
        pltpu.sync_copy(hist_sp.at[pl.ds(r0, ZN)], dbuf)
        def _rs(i, _):
          d = jnp.maximum(dbuf[pl.ds(i * L, L)], 1.0)
          dbuf[pl.ds(i * L, L)] = _rsqrt_newton(d)
          return 0
        lax.fori_loop(0, ZN // L, _rs, 0)
        pltpu.sync_copy(dbuf, out_hbm.at[pl.ds(r0, ZN)])
      plsc.subcore_barrier()


def _degrees(src_list, dst_list):
  kern = pl.kernel(
      _degrees_body,
      out_type=[jax.ShapeDtypeStruct((N,), jnp.float32) for _ in range(18)],
      mesh=_mesh,
      scratch_types=[
          pltpu.VMEM((SB,), jnp.int32),
          pltpu.VMEM((SB,), jnp.int32),
          pltpu.VMEM((SB,), jnp.float32),
          pltpu.VMEM((ZN,), jnp.float32),
          pltpu.VMEM((ZN,), jnp.float32),
          pltpu.VMEM_SHARED((N,), jnp.float32),
          pltpu.VMEM_SHARED((N,), jnp.float32),
      ],
  )
  outs = kern(*src_list, *dst_list)
  return outs[:9], outs[9:]


def _agg_body(*refs):
  y_refs = dict(zip(REL_NAMES, refs[0:9]))
  esrc_refs = dict(zip(REL_NAMES, refs[9:18]))
  edst_refs = dict(zip(REL_NAMES, refs[18:27]))
  id_refs = dict(zip(REL_NAMES, refs[27:36]))
  bias_ref = refs[36]
  out_refs = {"M": refs[37], "E": refs[38], "S": refs[39]}
  (srcbuf, dstbuf, fdst1r, gsrc, g0, gA, irbuf, bbuf, gsem0, gsem1,
   ssem0, ssem1, acc_sp) = refs[40:]
  gbufs = (g0, gA)
  gsems = (gsem0, gsem1)
  ssems = (ssem0, ssem1)

  cid = lax.axis_index("c")
  tid = lax.axis_index("s")

  pltpu.sync_copy(bias_ref, bbuf)
  iota = lax.iota(jnp.int32, L)
  dummy_dst = CHUNK + iota * 2 + (tid & 1)

  nwin = (NWIN // NS) + jnp.where(tid < NWIN - (NWIN // NS) * NS, 1, 0)

  def _chunk(kk, _c):
    ci = cid + NC * kk
    lo = ci * CHUNK
    hi = jnp.minimum(lo + CHUNK, N)
    climit = hi - lo

    for dname in ["M", "E", "S"]:
      di = ["M", "E", "S"].index(dname)
      out = out_refs[dname]
      bvs = [bbuf[di, pl.ds(c8 * L, L)] for c8 in range(D // L)]

      for k, rname in enumerate(DST_RELS[dname]):
        esrc = esrc_refs[rname]
        edst = edst_refs[rname]
        y_hbm = y_refs[rname]
        ir_hbm = id_refs[rname]

        def _zfill(i, _):
          for c8 in range(D // L):
            g0[i, pl.ds(c8 * L, L)] = jnp.zeros((L,), jnp.float32)
          return 0
        lax.fori_loop(0, WB, _zfill, 0)

        def _zero(w, _):
          pltpu.sync_copy(g0.at[pl.ds(0, WB)],
                          acc_sp.at[pl.ds((tid + NS * w) * WB, WB)])
          return 0
        lax.fori_loop(0, nwin, _zero, 0)
        plsc.subcore_barrier()

        for h in range(2):
          eoff = tid * EPT + h * EH
          pltpu.sync_copy(esrc.at[pl.ds(eoff, EH)], srcbuf)
          pltpu.sync_copy(edst.at[pl.ds(eoff, EH)], dstbuf)

          def _fill_gsrc(b, p):
            for o in range(0, SBE, L):
              gsrc[p, pl.ds(o, L)] = srcbuf[pl.ds(b * SBE + o, L)]

          def _fill_fdst(b, p):
            for o in range(0, SBE, L):
              dv = dstbuf[pl.ds(b * SBE + o, L)]
              m = (dv >= lo) & (dv < hi)
              fdst1r[p, pl.ds(o, L)] = jnp.where(m, dv - lo, dummy_dst)

          def _fire_gather(b, p):
            pltpu.async_copy(y_hbm.at[gsrc.at[p]], gbufs[p], gsems[p])

          _fill_gsrc(0, 0)
          _fire_gather(0, 0)
          _fill_gsrc(1, 1)
          _fire_gather(1, 1)

          def _pipe2(t, _):
            for p in range(2):
              b = 2 * t + p
              @pl.when(b < NSUB)
              def _():
                pltpu.make_async_copy(y_hbm.at[gsrc.at[p]], gbufs[p],
                                      gsems[p]).wait()
                _fill_fdst(b, p)
                pltpu.async_copy(gbufs[p], acc_sp.at[fdst1r.at[p]],
                                 ssems[p], add=True)
                @pl.when(b + 2 < NSUB)
                def _():
                  pltpu.make_async_copy(gbufs[p], acc_sp.at[fdst1r.at[p]],
                                        ssems[p]).wait()
                  _fill_gsrc(b + 2, p)
                  _fire_gather(b + 2, p)
            return 0
          lax.fori_loop(0, (NSUB + 1) // 2, _pipe2, 0)
          pltpu.make_async_copy(gbufs[0], acc_sp.at[fdst1r.at[0]],
                                ssems[(NSUB - 2) & 1]).wait()
          pltpu.make_async_copy(gbufs[0], acc_sp.at[fdst1r.at[0]],
                                ssems[(NSUB - 1) & 1]).wait()
        plsc.subcore_barrier()

        def _wb(w, _):
          ls = pl.multiple_of((tid + NS * w) * WB, 8)

          @pl.when(ls < climit)
          def _do_wb():
            pltpu.sync_copy(acc_sp.at[pl.ds(ls, WB)], g0.at[pl.ds(0, WB)])
            pltpu.sync_copy(ir_hbm.at[pl.ds(lo + ls, WB)],
                            irbuf.at[pl.ds(0, WB)])
            if k > 0:
              pltpu.sync_copy(out.at[pl.ds(lo + ls, WB)],
                              gA.at[pl.ds(0, WB)])

            def _row(i, _):
              gi = i >> 4
              li = i & (L - 1)
              irv = irbuf[pl.ds(gi * L, L)]
              s = _lane_bcast_dyn(irv, li)
              for c8 in range(D // L):
                v = g0[i, pl.ds(c8 * L, L)] * s
                if k > 0:
                  v = v + gA[i, pl.ds(c8 * L, L)]
                if k == 2:
                  v = jnp.maximum(v + bvs[c8], 0.0)
                g0[i, pl.ds(c8 * L, L)] = v
              return 0
            lax.fori_loop(0, WB, _row, 0)
            pltpu.sync_copy(g0.at[pl.ds(0, WB)], out.at[pl.ds(lo + ls, WB)])
          return 0
        lax.fori_loop(0, nwin, _wb, 0)
        plsc.subcore_barrier()
    return 0

  lax.fori_loop(0, 2, _chunk, 0)


def _aggregate(y_by_rel, src_list, dst_list, id_list, bias_sums):
  kern = pl.kernel(
      _agg_body,
      out_type=[jax.ShapeDtypeStruct((N, D), jnp.float32) for _ in range(3)],
      mesh=_mesh,
      scratch_types=[
          pltpu.VMEM((EH,), jnp.int32),
          pltpu.VMEM((EH,), jnp.int32),
          pltpu.VMEM((2, SBE), jnp.int32),
          pltpu.VMEM((2, SBE), jnp.int32),
          pltpu.VMEM((SBE, D), jnp.float32),
          pltpu.VMEM((SBE, D), jnp.float32),
          pltpu.VMEM((WB + 8, ), jnp.float32),
          pltpu.VMEM((3, D), jnp.float32),
          pltpu.SemaphoreType.DMA,
          pltpu.SemaphoreType.DMA,
          pltpu.SemaphoreType.DMA,
          pltpu.SemaphoreType.DMA,
          pltpu.VMEM_SHARED((ACC_ROWS, D), jnp.float32),
      ],
  )
  return kern(*[y_by_rel[n] for n in REL_NAMES], *src_list, *dst_list,
              *id_list, bias_sums)


def _mm_body(x_ref, w0_ref, w1_ref, w2_ref, d0_ref, d1_ref, d2_ref,
             o0_ref, o1_ref, o2_ref):
  x = x_ref[...]
  o0_ref[...] = jnp.dot(x * d0_ref[...][:, None], w0_ref[...],
                        preferred_element_type=jnp.float32)
  o1_ref[...] = jnp.dot(x * d1_ref[...][:, None], w1_ref[...],
                        preferred_element_type=jnp.float32)
  o2_ref[...] = jnp.dot(x * d2_ref[...][:, None], w2_ref[...],
                        preferred_element_type=jnp.float32)


def _matmul3(x, ws, ods):
  bm = 512
  grid = (pl.cdiv(N, bm),)
  wspec = pl.BlockSpec((D, D), lambda i: (0, 0))
  dspec = pl.BlockSpec((bm,), lambda i: (i,))
  ospec = pl.BlockSpec((bm, D), lambda i: (i, 0))
  return pl.pallas_call(
      _mm_body,
      grid=grid,
      in_specs=[pl.BlockSpec((bm, D), lambda i: (i, 0)),
                wspec, wspec, wspec, dspec, dspec, dspec],
      out_specs=[ospec, ospec, ospec],
      out_shape=[jax.ShapeDtypeStruct((N, D), jnp.float32) for _ in range(3)],
  )(x, *ws, *ods)


def kernel(x_M, x_E, x_S,
           ei_M_M, ei_M_E, ei_M_S, ei_E_S, ei_E_M, ei_S_M, ei_S_E, ei_S_S,
           ei_E_E,
           W_M_M, W_M_E, W_M_S, W_E_S, W_E_M, W_S_M, W_S_E, W_S_S, W_E_E,
           b_M_M, b_M_E, b_M_S, b_E_S, b_E_M, b_S_M, b_S_E, b_S_S, b_E_E):
  x = {"M": x_M, "E": x_E, "S": x_S}
  ei = {"M_M": ei_M_M, "M_E": ei_M_E, "M_S": ei_M_S, "E_S": ei_E_S,
        "E_M": ei_E_M, "S_M": ei_S_M, "S_E": ei_S_E, "S_S": ei_S_S,
        "E_E": ei_E_E}
  W = {"M_M": W_M_M, "M_E": W_M_E, "M_S": W_M_S, "E_S": W_E_S, "E_M": W_E_M,
       "S_M": W_S_M, "S_E": W_S_E, "S_S": W_S_S, "E_E": W_E_E}
  b = {"M_M": b_M_M, "M_E": b_M_E, "M_S": b_M_S, "E_S": b_E_S, "E_M": b_E_M,
       "S_M": b_S_M, "S_E": b_S_E, "S_S": b_S_S, "E_E": b_E_E}

  src_list = [ei[n][0] for n in REL_NAMES]
  dst_list = [ei[n][1] for n in REL_NAMES]
  od_list, id_list = _degrees(src_list, dst_list)
  od = dict(zip(REL_NAMES, od_list))

  y_by_rel = {}
  for s, slots in SRC_SLOTS.items():
    ys = _matmul3(x[s], [W[r] for r in slots], [od[r] for r in slots])
    for r, y in zip(slots, ys):
      y_by_rel[r] = y

  bias_sums = jnp.stack([b[DST_RELS[d][0]] + b[DST_RELS[d][1]]
                         + b[DST_RELS[d][2]] for d in ["M", "E", "S"]])

  return tuple(_aggregate(y_by_rel, src_list, dst_list, id_list,
                          bias_sums))
